# baseline (speedup 1.0000x reference)
import jax
import jax.numpy as jnp
from jax.experimental import pallas as pl
from jax.experimental.pallas import tpu as pltpu

_N_TOKENS = 32768
_D_MODEL = 2048
_NUM_EXPERTS = 64
_TOP_K = 8
_BT = 512
_NBLK = _N_TOKENS // _BT
_NSTEP = _NBLK // 4
_NSLOT = 8


def _top8_softmax(vals, ow_ref, oi_ref, row0):
    iota = jax.lax.broadcasted_iota(jnp.int32, vals.shape, 0)
    top_vals = []
    top_idxs = []
    for _ in range(_TOP_K):
        m = jnp.max(vals, axis=0, keepdims=True)
        idx = jnp.min(jnp.where(vals == m, iota, _NUM_EXPERTS), axis=0,
                      keepdims=True)
        top_vals.append(m)
        top_idxs.append(idx)
        vals = jnp.where(iota == idx, -jnp.inf, vals)
    tv = jnp.concatenate(top_vals, axis=0)
    ti = jnp.concatenate(top_idxs, axis=0)
    e = jnp.exp(tv - tv[0:1])
    wgt = e / jnp.sum(e, axis=0, keepdims=True)
    eye = jnp.eye(_TOP_K, dtype=jnp.float32)
    ow_ref[row0:row0 + _BT, :] = jax.lax.dot_general(
        wgt, eye, (((0,), (0,)), ((), ())),
        preferred_element_type=jnp.float32)
    ti_f = ti.astype(jnp.float32)
    oi_ref[row0:row0 + _BT, :] = jax.lax.dot_general(
        ti_f, eye, (((0,), (0,)), ((), ())),
        preferred_element_type=jnp.float32).astype(jnp.int32)


def _gate_body(x_hbm, w_ref, wp1_ref, ip1_ref, wp2_ref, ip2_ref,
               wt_ref, it_ref, xbuf, lbuf, sems):
    j = pl.program_id(0)

    def cp(blk, slot):
        return pltpu.make_async_copy(
            x_hbm.at[pl.ds(blk * _BT, _BT), :], xbuf.at[slot], sems.at[slot])

    @pl.when(j == 0)
    def _prime():
        for b in range(4):
            cp(b, b).start()

    @pl.when(j < _NSTEP - 1)
    def _prefetch():
        for b in range(4):
            nxt = (j + 1) * 4 + b
            cp(nxt, nxt % _NSLOT).start()

    base = j * 4
    for b in range(4):
        cp(base + b, (base + b) % _NSLOT).wait()

    w = w_ref[...]

    def mm(b, lslot):
        lbuf[lslot] = jax.lax.dot_general(
            w, xbuf[(base + b) % _NSLOT], (((1,), (1,)), ((), ())),
            preferred_element_type=jnp.float32,
        )

    _top8_softmax(lbuf[2], wp1_ref, ip1_ref, 0)
    _top8_softmax(lbuf[3], wp1_ref, ip1_ref, _BT)
    mm(0, 0)
    mm(1, 1)
    _top8_softmax(lbuf[0], wp2_ref, ip2_ref, 0)
    _top8_softmax(lbuf[1], wp2_ref, ip2_ref, _BT)
    mm(2, 2)
    mm(3, 3)

    @pl.when(j == _NSTEP - 1)
    def _tail():
        _top8_softmax(lbuf[2], wt_ref, it_ref, 0)
        _top8_softmax(lbuf[3], wt_ref, it_ref, _BT)


@jax.jit
def kernel(x, W):
    p1 = _NSTEP - 1
    outs = pl.pallas_call(
        _gate_body,
        grid=(_NSTEP,),
        in_specs=[
            pl.BlockSpec(memory_space=pl.ANY),
            pl.BlockSpec((_NUM_EXPERTS, _D_MODEL), lambda j: (0, 0)),
        ],
        out_specs=[
            pl.BlockSpec((2 * _BT, _TOP_K), lambda j: (jnp.maximum(j - 1, 0), 0)),
            pl.BlockSpec((2 * _BT, _TOP_K), lambda j: (jnp.maximum(j - 1, 0), 0)),
            pl.BlockSpec((2 * _BT, _TOP_K), lambda j: (j, 0)),
            pl.BlockSpec((2 * _BT, _TOP_K), lambda j: (j, 0)),
            pl.BlockSpec((2 * _BT, _TOP_K), lambda j: (0, 0)),
            pl.BlockSpec((2 * _BT, _TOP_K), lambda j: (0, 0)),
        ],
        out_shape=[
            jax.ShapeDtypeStruct((p1 * 2 * _BT, _TOP_K), jnp.float32),
            jax.ShapeDtypeStruct((p1 * 2 * _BT, _TOP_K), jnp.int32),
            jax.ShapeDtypeStruct((_NSTEP * 2 * _BT, _TOP_K), jnp.float32),
            jax.ShapeDtypeStruct((_NSTEP * 2 * _BT, _TOP_K), jnp.int32),
            jax.ShapeDtypeStruct((2 * _BT, _TOP_K), jnp.float32),
            jax.ShapeDtypeStruct((2 * _BT, _TOP_K), jnp.int32),
        ],
        scratch_shapes=[
            pltpu.VMEM((_NSLOT, _BT, _D_MODEL), jnp.float32),
            pltpu.VMEM((4, _NUM_EXPERTS, _BT), jnp.float32),
            pltpu.SemaphoreType.DMA((_NSLOT,)),
        ],
    )(x, W)
    wp1, ip1, wp2, ip2, wt, it = outs

    def assemble(even, odd_main, odd_tail):
        ev = even.reshape(_NSTEP, 2 * _BT, _TOP_K)
        od = jnp.concatenate(
            [odd_main.reshape(p1, 2 * _BT, _TOP_K),
             odd_tail.reshape(1, 2 * _BT, _TOP_K)], axis=0)
        return jnp.stack([ev, od], axis=1).reshape(_N_TOKENS, _TOP_K)

    return (assemble(wp2, wp1, wt), assemble(ip2, ip1, it))


# speedup vs baseline: 1.0778x; 1.0778x over previous
import jax
import jax.numpy as jnp
from jax.experimental import pallas as pl
from jax.experimental.pallas import tpu as pltpu

_N_TOKENS = 32768
_D_MODEL = 2048
_NUM_EXPERTS = 64
_TOP_K = 8
_BT = 1024
_NBUF = 4


def _top8_softmax(vals, out_w_ref, out_i_ref):
    iota = jax.lax.broadcasted_iota(jnp.int32, vals.shape, 0)
    top_vals = []
    top_idxs = []
    for _ in range(_TOP_K):
        m = jnp.max(vals, axis=0, keepdims=True)
        idx = jnp.min(jnp.where(vals == m, iota, _NUM_EXPERTS), axis=0,
                      keepdims=True)
        top_vals.append(m)
        top_idxs.append(idx)
        vals = jnp.where(iota == idx, -jnp.inf, vals)
    tv = jnp.concatenate(top_vals, axis=0)
    ti = jnp.concatenate(top_idxs, axis=0)
    e = jnp.exp(tv - tv[0:1])
    wgt = e / jnp.sum(e, axis=0, keepdims=True)
    eye = jnp.eye(_TOP_K, dtype=jnp.float32)
    out_w_ref[...] = jax.lax.dot_general(
        wgt, eye, (((0,), (0,)), ((), ())),
        preferred_element_type=jnp.float32)
    ti_f = ti.astype(jnp.float32)
    out_i_ref[...] = jax.lax.dot_general(
        ti_f, eye, (((0,), (0,)), ((), ())),
        preferred_element_type=jnp.float32).astype(jnp.int32)


def _gate_body(x_hbm, w_ref, ow_ref, oi_ref, xbuf, sems):
    i = pl.program_id(0)
    nblk = pl.num_programs(0)

    def cp(blk, slot):
        return pltpu.make_async_copy(
            x_hbm.at[pl.ds(blk * _BT, _BT), :], xbuf.at[slot], sems.at[slot])

    @pl.when(i == 0)
    def _prime():
        for b in range(_NBUF - 1):
            cp(b, b).start()

    nxt = i + _NBUF - 1
    @pl.when(nxt < nblk)
    def _prefetch():
        cp(nxt, nxt % _NBUF).start()

    slot = i % _NBUF
    cp(i, slot).wait()
    vals = jax.lax.dot_general(
        w_ref[...], xbuf[slot], (((1,), (1,)), ((), ())),
        preferred_element_type=jnp.float32,
    )
    _top8_softmax(vals, ow_ref, oi_ref)


@jax.jit
def kernel(x, W):
    grid = (_N_TOKENS // _BT,)
    return pl.pallas_call(
        _gate_body,
        grid=grid,
        in_specs=[
            pl.BlockSpec(memory_space=pl.ANY),
            pl.BlockSpec((_NUM_EXPERTS, _D_MODEL), lambda i: (0, 0)),
        ],
        out_specs=[
            pl.BlockSpec((_BT, _TOP_K), lambda i: (i, 0)),
            pl.BlockSpec((_BT, _TOP_K), lambda i: (i, 0)),
        ],
        out_shape=[
            jax.ShapeDtypeStruct((_N_TOKENS, _TOP_K), jnp.float32),
            jax.ShapeDtypeStruct((_N_TOKENS, _TOP_K), jnp.int32),
        ],
        scratch_shapes=[
            pltpu.VMEM((_NBUF, _BT, _D_MODEL), jnp.float32),
            pltpu.SemaphoreType.DMA((_NBUF,)),
        ],
    )(x, W)
